# baseline (speedup 1.0000x reference)
import functools

import jax
import jax.numpy as jnp
from jax import lax
from jax.experimental import pallas as pl
from jax.experimental.pallas import tpu as pltpu
from jax.experimental.pallas import tpu_sc as plsc

VOCAB = 100000
EMBED_DIM = 128
NUM_CLASS = 4
BATCH = 4096
BAG_LEN = 50

LANES = 16
NUM_WORKERS = 32
BPW = BATCH // NUM_WORKERS

PACK = EMBED_DIM // LANES
VLIN = VOCAB // PACK
LIN_BLOCK = 2504


def _proj_body(emb4_ref, w5_ref, out_ref):
    acc = jnp.dot(
        emb4_ref[0], w5_ref[0], preferred_element_type=jnp.float32
    )
    for m in range(1, PACK):
        acc += jnp.dot(
            emb4_ref[m], w5_ref[m], preferred_element_type=jnp.float32
        )
    out_ref[...] = acc


def _project(emb4, w5):
    grid = (VLIN + LIN_BLOCK - 1) // LIN_BLOCK
    return pl.pallas_call(
        _proj_body,
        grid=(grid,),
        in_specs=[
            pl.BlockSpec((PACK, LIN_BLOCK, EMBED_DIM), lambda i: (0, i, 0)),
            pl.BlockSpec((PACK, EMBED_DIM, EMBED_DIM), lambda i: (0, 0, 0)),
        ],
        out_specs=pl.BlockSpec((LIN_BLOCK, EMBED_DIM), lambda i: (i, 0)),
        out_shape=jax.ShapeDtypeStruct((VLIN, EMBED_DIM), jnp.float32),
    )(emb4, w5)


_SC_MESH = plsc.VectorSubcoreMesh(
    core_axis_name="c", subcore_axis_name="s", num_cores=2, num_subcores=16
)

_CHUNK = 10


@functools.partial(
    pl.kernel,
    out_type=jax.ShapeDtypeStruct((BATCH, LANES), jnp.float32),
    mesh=_SC_MESH,
    compiler_params=pltpu.CompilerParams(use_tc_tiling_on_sc=False),
    scratch_types=[
        pltpu.VMEM((BAG_LEN, BPW), jnp.int32),
        pltpu.VMEM((BPW, LANES), jnp.float32),
        pltpu.SemaphoreType.DMA,
        pltpu.SemaphoreType.DMA,
    ],
)
def _bag_sum(p_hbm, idx_hbm, bias_hbm, out_hbm, idx_v, out_v, isem, sem):
    wid = lax.axis_index("s") * 2 + lax.axis_index("c")

    pltpu.async_copy(idx_hbm.at[wid], idx_v, isem)
    biascp = pltpu.make_async_copy(bias_hbm, out_v, sem)
    biascp.start()
    pltpu.make_async_copy(idx_hbm.at[wid], idx_v, isem).wait()
    biascp.wait()

    for k in range(_CHUNK):
        pltpu.async_copy(p_hbm.at[idx_v.at[k]], out_v, sem, add=True)

    @pl.loop(1, BAG_LEN // _CHUNK)
    def _round(ci):
        base = ci * _CHUNK
        for k in range(_CHUNK):
            pltpu.async_copy(
                p_hbm.at[idx_v.at[base + k]], out_v, sem, add=True
            )
        for _ in range(_CHUNK):
            pltpu.make_async_copy(p_hbm.at[idx_v.at[0]], out_v, sem).wait()

    for _ in range(_CHUNK):
        pltpu.make_async_copy(p_hbm.at[idx_v.at[0]], out_v, sem).wait()

    pltpu.sync_copy(out_v, out_hbm.at[pl.ds(wid * BPW, BPW)])


def kernel(text, emb_weight, fc_w, fc_b):
    ws = fc_w.T * (1.0 / BAG_LEN)
    w5 = jnp.zeros((PACK, EMBED_DIM, EMBED_DIM), jnp.float32)
    for j in range(PACK):
        w5 = w5.at[j, :, j * LANES : j * LANES + NUM_CLASS].set(ws)
    bias2d = jnp.broadcast_to(
        jnp.zeros((LANES,), jnp.float32).at[:NUM_CLASS].set(fc_b), (BPW, LANES)
    )
    tt = (text % VLIN) * PACK + text // VLIN
    idx = tt.reshape(NUM_WORKERS, BPW, BAG_LEN).transpose(0, 2, 1)

    emb4 = emb_weight.reshape(PACK, VLIN, EMBED_DIM)
    p_lin = _project(emb4, w5)
    p = p_lin.reshape(VOCAB, LANES)
    out16 = _bag_sum(p, idx, bias2d)
    return out16[:, :NUM_CLASS]


# speedup vs baseline: 1.0150x; 1.0150x over previous
import functools

import jax
import jax.numpy as jnp
from jax import lax
from jax.experimental import pallas as pl
from jax.experimental.pallas import tpu as pltpu
from jax.experimental.pallas import tpu_sc as plsc

VOCAB = 100000
EMBED_DIM = 128
NUM_CLASS = 4
BATCH = 4096
BAG_LEN = 50

LANES = 16
NUM_WORKERS = 32
BPW = BATCH // NUM_WORKERS

PACK = EMBED_DIM // LANES
VLIN = VOCAB // PACK
LIN_BLOCK = 2504


def _proj_body(emb4_ref, w5_ref, out_ref):
    acc = jnp.dot(
        emb4_ref[0].astype(jnp.bfloat16),
        w5_ref[0],
        preferred_element_type=jnp.float32,
    )
    for m in range(1, PACK):
        acc += jnp.dot(
            emb4_ref[m].astype(jnp.bfloat16),
            w5_ref[m],
            preferred_element_type=jnp.float32,
        )
    out_ref[...] = acc


def _project(emb4, w5):
    grid = (VLIN + LIN_BLOCK - 1) // LIN_BLOCK
    return pl.pallas_call(
        _proj_body,
        grid=(grid,),
        in_specs=[
            pl.BlockSpec((PACK, LIN_BLOCK, EMBED_DIM), lambda i: (0, i, 0)),
            pl.BlockSpec((PACK, EMBED_DIM, EMBED_DIM), lambda i: (0, 0, 0)),
        ],
        out_specs=pl.BlockSpec((LIN_BLOCK, EMBED_DIM), lambda i: (i, 0)),
        out_shape=jax.ShapeDtypeStruct((VLIN, EMBED_DIM), jnp.float32),
    )(emb4, w5)


_SC_MESH = plsc.VectorSubcoreMesh(
    core_axis_name="c", subcore_axis_name="s", num_cores=2, num_subcores=16
)

_CHUNK = 10


@functools.partial(
    pl.kernel,
    out_type=jax.ShapeDtypeStruct((BATCH, LANES), jnp.float32),
    mesh=_SC_MESH,
    compiler_params=pltpu.CompilerParams(use_tc_tiling_on_sc=False),
    scratch_types=[
        pltpu.VMEM((BAG_LEN, BPW), jnp.int32),
        pltpu.VMEM((BPW, LANES), jnp.float32),
        pltpu.SemaphoreType.DMA,
        pltpu.SemaphoreType.DMA,
    ],
)
def _bag_sum(p_hbm, idx_hbm, bias_hbm, out_hbm, idx_v, out_v, isem, sem):
    wid = lax.axis_index("s") * 2 + lax.axis_index("c")

    pltpu.async_copy(idx_hbm.at[wid], idx_v, isem)
    biascp = pltpu.make_async_copy(bias_hbm, out_v, sem)
    biascp.start()
    pltpu.make_async_copy(idx_hbm.at[wid], idx_v, isem).wait()
    biascp.wait()

    for k in range(_CHUNK):
        pltpu.async_copy(p_hbm.at[idx_v.at[k]], out_v, sem, add=True)

    @pl.loop(1, BAG_LEN // _CHUNK)
    def _round(ci):
        base = ci * _CHUNK
        for k in range(_CHUNK):
            pltpu.async_copy(
                p_hbm.at[idx_v.at[base + k]], out_v, sem, add=True
            )
        for _ in range(_CHUNK):
            pltpu.make_async_copy(p_hbm.at[idx_v.at[0]], out_v, sem).wait()

    for _ in range(_CHUNK):
        pltpu.make_async_copy(p_hbm.at[idx_v.at[0]], out_v, sem).wait()

    pltpu.sync_copy(out_v, out_hbm.at[pl.ds(wid * BPW, BPW)])


def kernel(text, emb_weight, fc_w, fc_b):
    ws = (fc_w.T * (1.0 / BAG_LEN)).astype(jnp.bfloat16)
    w5 = jnp.zeros((PACK, EMBED_DIM, EMBED_DIM), jnp.bfloat16)
    for j in range(PACK):
        w5 = w5.at[j, :, j * LANES : j * LANES + NUM_CLASS].set(ws)
    bias2d = jnp.broadcast_to(
        jnp.zeros((LANES,), jnp.float32).at[:NUM_CLASS].set(fc_b), (BPW, LANES)
    )
    tt = (text % VLIN) * PACK + text // VLIN
    idx = tt.reshape(NUM_WORKERS, BPW, BAG_LEN).transpose(0, 2, 1)

    emb4 = emb_weight.reshape(PACK, VLIN, EMBED_DIM)
    p_lin = _project(emb4, w5)
    p = p_lin.reshape(VOCAB, LANES)
    out16 = _bag_sum(p, idx, bias2d)
    return out16[:, :NUM_CLASS]


# speedup vs baseline: 1.4619x; 1.4404x over previous
import functools

import jax
import jax.numpy as jnp
from jax import lax
from jax.experimental import pallas as pl
from jax.experimental.pallas import tpu as pltpu
from jax.experimental.pallas import tpu_sc as plsc

VOCAB = 100000
EMBED_DIM = 128
NUM_CLASS = 4
BATCH = 4096
BAG_LEN = 50

LANES = 16
NUM_WORKERS = 32
BPW = BATCH // NUM_WORKERS

PACK = EMBED_DIM // LANES
EMB_BLOCK = 8192


def _proj_body(emb_ref, w_ref, out_ref):
    out_ref[:, 0:LANES] = jnp.dot(
        emb_ref[...], w_ref[...], preferred_element_type=jnp.float32
    )


def _project(emb_weight, w_pad):
    grid = (VOCAB + EMB_BLOCK - 1) // EMB_BLOCK
    return pl.pallas_call(
        _proj_body,
        grid=(grid,),
        in_specs=[
            pl.BlockSpec((EMB_BLOCK, EMBED_DIM), lambda i: (i, 0)),
            pl.BlockSpec((EMBED_DIM, LANES), lambda i: (0, 0)),
        ],
        out_specs=pl.BlockSpec((EMB_BLOCK, EMBED_DIM), lambda i: (i, 0)),
        out_shape=jax.ShapeDtypeStruct((VOCAB, EMBED_DIM), jnp.float32),
    )(emb_weight, w_pad)


_SC_MESH = plsc.VectorSubcoreMesh(
    core_axis_name="c", subcore_axis_name="s", num_cores=2, num_subcores=16
)

_CHUNK = 25


@functools.partial(
    pl.kernel,
    out_type=jax.ShapeDtypeStruct((BATCH, LANES), jnp.float32),
    mesh=_SC_MESH,
    compiler_params=pltpu.CompilerParams(use_tc_tiling_on_sc=False),
    scratch_types=[
        pltpu.VMEM((BAG_LEN, BPW), jnp.int32),
        pltpu.VMEM((BPW, LANES), jnp.float32),
        pltpu.SemaphoreType.DMA,
        pltpu.SemaphoreType.DMA,
    ],
)
def _bag_sum(p_hbm, idx_hbm, bias_hbm, out_hbm, idx_v, out_v, isem, sem):
    wid = lax.axis_index("s") * 2 + lax.axis_index("c")

    pltpu.async_copy(idx_hbm.at[wid], idx_v, isem)
    biascp = pltpu.make_async_copy(bias_hbm, out_v, sem)
    biascp.start()
    pltpu.make_async_copy(idx_hbm.at[wid], idx_v, isem).wait()
    biascp.wait()

    for k in range(_CHUNK):
        pltpu.async_copy(p_hbm.at[idx_v.at[k]], out_v, sem, add=True)

    @pl.loop(1, BAG_LEN // _CHUNK)
    def _round(ci):
        base = ci * _CHUNK
        for k in range(_CHUNK):
            pltpu.async_copy(
                p_hbm.at[idx_v.at[base + k]], out_v, sem, add=True
            )
        for _ in range(_CHUNK):
            pltpu.make_async_copy(p_hbm.at[idx_v.at[0]], out_v, sem).wait()

    for _ in range(_CHUNK):
        pltpu.make_async_copy(p_hbm.at[idx_v.at[0]], out_v, sem).wait()

    pltpu.sync_copy(out_v, out_hbm.at[pl.ds(wid * BPW, BPW)])


def kernel(text, emb_weight, fc_w, fc_b):
    w_pad = (
        jnp.zeros((EMBED_DIM, LANES), jnp.float32)
        .at[:, :NUM_CLASS]
        .set(fc_w.T * (1.0 / BAG_LEN))
    )
    bias2d = jnp.broadcast_to(
        jnp.zeros((LANES,), jnp.float32).at[:NUM_CLASS].set(fc_b), (BPW, LANES)
    )
    tt = text * PACK
    idx = tt.reshape(NUM_WORKERS, BPW, BAG_LEN).transpose(0, 2, 1)

    p_wide = _project(emb_weight, w_pad)
    p = p_wide.reshape(VOCAB * PACK, LANES)
    out16 = _bag_sum(p, idx, bias2d)
    return out16[:, :NUM_CLASS]
